# Initial kernel scaffold; baseline (speedup 1.0000x reference)
import jax
import jax.numpy as jnp
from jax.experimental import pallas as pl


def kernel(x, edge_index, W, b):
    raise NotImplementedError("write your pallas kernel here")


# speedup vs baseline: 14.9100x; 14.9100x over previous
import functools

import jax
import jax.numpy as jnp
from jax import lax
from jax.experimental import pallas as pl
from jax.experimental.pallas import tpu as pltpu
from jax.experimental.pallas import tpu_sc as plsc

NC = 2
NS = 16
NW = NC * NS
CHUNK = 128
EB = 2048
NF = 128
TC_BLK = 1280


def _make_agg_kernel(n_pad, n_chunks, d):
    rows = n_pad // NS

    @functools.partial(
        pl.kernel,
        out_type=jax.ShapeDtypeStruct((NC, n_pad, d), jnp.float32),
        mesh=plsc.VectorSubcoreMesh(core_axis_name="c", subcore_axis_name="s"),
        scratch_types=[
            pltpu.VMEM((n_chunks, CHUNK), jnp.int32),
            pltpu.VMEM((n_chunks, CHUNK), jnp.int32),
            pltpu.VMEM((CHUNK, d), jnp.float32),
            pltpu.VMEM_SHARED((n_pad, d), jnp.float32),
            pltpu.SemaphoreType.DMA,
        ],
    )
    def agg_kernel(g_hbm, src_hbm, dst_hbm, zrows_hbm, part_hbm,
                   src_v, dst_v, rows_v, acc, sem):
        c = lax.axis_index("c")
        s = lax.axis_index("s")
        wid = c * NS + s
        pltpu.sync_copy(src_hbm.at[wid], src_v)
        pltpu.sync_copy(dst_hbm.at[wid], dst_v)
        pltpu.sync_copy(zrows_hbm, acc.at[pl.ds(s * rows, rows)])
        plsc.subcore_barrier()

        def body(j, carry):
            pltpu.async_copy(g_hbm.at[src_v.at[j]], rows_v, sem).wait()
            pltpu.sync_copy(rows_v, acc.at[dst_v.at[j]], add=True)
            return carry

        lax.fori_loop(0, n_chunks, body, 0)
        plsc.subcore_barrier()
        pltpu.sync_copy(acc.at[pl.ds(s * rows, rows)],
                        part_hbm.at[c, pl.ds(s * rows, rows)])

    return agg_kernel


def _make_hist_body(nc_bins):
    def _hist_body(dst_ref, o_ref):
        i = pl.program_id(0)

        @pl.when(i == 0)
        def _():
            o_ref[...] = jnp.zeros_like(o_ref)

        d = dst_ref[0]
        cid = lax.broadcasted_iota(jnp.int32, (nc_bins, 1), 0)
        fid = lax.broadcasted_iota(jnp.int32, (NF, 1), 0)
        oc = ((d >> 7) == cid).astype(jnp.float32)
        of = ((d & (NF - 1)) == fid).astype(jnp.float32)
        o_ref[...] += lax.dot_general(
            oc, of, (((1,), (1,)), ((), ())),
            preferred_element_type=jnp.float32)

    return _hist_body


def _g_body(x_ref, w_ref, deg_ref, g_ref):
    dis = lax.rsqrt(1.0 + deg_ref[...])
    g_ref[...] = jnp.dot(x_ref[...], w_ref[...],
                         preferred_element_type=jnp.float32) * dis


def _out_body(parts_ref, g_ref, deg_ref, b_ref, o_ref):
    dis = lax.rsqrt(1.0 + deg_ref[...])
    o_ref[...] = dis * (parts_ref[0] + parts_ref[1] + g_ref[...]) + b_ref[...]


def kernel(x, edge_index, W, b):
    n, d = x.shape
    e = edge_index.shape[1]
    n_pad = -(-n // TC_BLK) * TC_BLK
    if n_pad == n:
        n_pad += TC_BLK
    per_tile = -(-e // (NW * CHUNK)) * CHUNK
    n_chunks = per_tile // CHUNK
    e_pad = NW * per_tile
    n_eblk = e_pad // EB

    ei = edge_index.astype(jnp.int32)
    pad = jnp.full((e_pad - e,), n, jnp.int32)
    src = jnp.concatenate([ei[0], pad]).reshape(NW, n_chunks, CHUNK)
    dst_flat = jnp.concatenate([ei[1], pad])
    dst = dst_flat.reshape(NW, n_chunks, CHUNK)
    x_pad = jnp.pad(x, ((0, n_pad - n), (0, 0)))
    z_rows = jnp.zeros((n_pad // NS, d), jnp.float32)

    nc_bins = n_pad // NF
    hist = pl.pallas_call(
        _make_hist_body(nc_bins),
        grid=(n_eblk,),
        in_specs=[pl.BlockSpec((1, 1, EB), lambda i: (i, 0, 0))],
        out_specs=pl.BlockSpec((nc_bins, NF), lambda i: (0, 0)),
        out_shape=jax.ShapeDtypeStruct((nc_bins, NF), jnp.float32),
    )(dst_flat.reshape(n_eblk, 1, EB))
    deg_col = hist.reshape(n_pad, 1)

    grid = (n_pad // TC_BLK,)
    g = pl.pallas_call(
        _g_body,
        grid=grid,
        in_specs=[
            pl.BlockSpec((TC_BLK, d), lambda i: (i, 0)),
            pl.BlockSpec((d, d), lambda i: (0, 0)),
            pl.BlockSpec((TC_BLK, 1), lambda i: (i, 0)),
        ],
        out_specs=pl.BlockSpec((TC_BLK, d), lambda i: (i, 0)),
        out_shape=jax.ShapeDtypeStruct((n_pad, d), jnp.float32),
    )(x_pad, W, deg_col)

    parts = _make_agg_kernel(n_pad, n_chunks, d)(g, src, dst, z_rows)

    out_full = pl.pallas_call(
        _out_body,
        grid=grid,
        in_specs=[
            pl.BlockSpec((NC, TC_BLK, d), lambda i: (0, i, 0)),
            pl.BlockSpec((TC_BLK, d), lambda i: (i, 0)),
            pl.BlockSpec((TC_BLK, 1), lambda i: (i, 0)),
            pl.BlockSpec((1, d), lambda i: (0, 0)),
        ],
        out_specs=pl.BlockSpec((TC_BLK, d), lambda i: (i, 0)),
        out_shape=jax.ShapeDtypeStruct((n_pad, d), jnp.float32),
    )(parts, g, deg_col, b.reshape(1, d))

    return out_full[:n]
